# baseline (speedup 1.0000x reference)
import jax
import jax.numpy as jnp
from jax.experimental import pallas as pl
from jax.experimental.pallas import tpu as pltpu

_N = 100000
_C = 80
_ALPHA = 0.25
_LOSS_WEIGHT = 1.0
_BR = 2000


def _focal_kernel(pred_ref, t0_ref, t1_ref, w0_ref, w1_ref, out_ref):
    i = pl.program_id(0)
    x = pred_ref[...]
    e = jnp.exp(-jnp.abs(x))
    log_p = jnp.minimum(x, 0.0) - jnp.log1p(e)
    log_1mp = log_p - x
    r = 1.0 / (1.0 + e)
    p = jnp.where(x >= 0, r, e * r)
    omp = 1.0 - p
    pos = (-_ALPHA) * (omp * omp) * log_p
    neg = (-(1.0 - _ALPHA)) * (p * p) * log_1mp

    t0 = t0_ref[...]
    t1 = t1_ref[...]
    w0 = w0_ref[...]
    w1 = w1_ref[...]
    cols = jax.lax.broadcasted_iota(jnp.int32, x.shape, 1)
    e0 = t0 == cols
    e1 = t1 == cols
    only1 = (t0 < _C) & (t1 == _C)
    only2 = (t1 < _C) & (t0 == _C)
    qbase = jnp.where(only1, w0, jnp.where(only2, w1, 0.5 * (w0 + w1)))
    coef_p = jnp.where(e1, w1, jnp.where(e0, w0, 0.0))
    coef_n = jnp.where(e0 | e1, 0.0, qbase)
    loss = coef_p * pos + coef_n * neg

    @pl.when(i == 0)
    def _():
        out_ref[...] = jnp.zeros_like(out_ref)

    out_ref[...] += jnp.sum(loss, axis=(0, 1), keepdims=True)


def kernel(pred, target0, target1, weight0, weight1):
    n, c = pred.shape
    grid = n // _BR
    t0 = target0.reshape(n, 1)
    t1 = target1.reshape(n, 1)
    w0 = weight0.reshape(n, 1)
    w1 = weight1.reshape(n, 1)
    out = pl.pallas_call(
        _focal_kernel,
        grid=(grid,),
        in_specs=[
            pl.BlockSpec((_BR, c), lambda i: (i, 0)),
            pl.BlockSpec((_BR, 1), lambda i: (i, 0)),
            pl.BlockSpec((_BR, 1), lambda i: (i, 0)),
            pl.BlockSpec((_BR, 1), lambda i: (i, 0)),
            pl.BlockSpec((_BR, 1), lambda i: (i, 0)),
        ],
        out_specs=pl.BlockSpec((1, 1), lambda i: (0, 0)),
        out_shape=jax.ShapeDtypeStruct((1, 1), jnp.float32),
        compiler_params=pltpu.CompilerParams(
            dimension_semantics=("arbitrary",),
        ),
    )(pred, t0, t1, w0, w1)
    return out[0, 0] * (_LOSS_WEIGHT / (n * c))


# speedup vs baseline: 1.4386x; 1.4386x over previous
import numpy as np
import jax
import jax.numpy as jnp
from jax.experimental import pallas as pl
from jax.experimental.pallas import tpu as pltpu

_N = 100000
_C = 80
_ALPHA = 0.25
_LOSS_WEIGHT = 1.0
_G = 25
_R = _N // (_G * 8)
_L = 8 * _C

_E_NP = np.zeros((8, _L), np.float32)
for _j in range(8):
    _E_NP[_j, _j * _C:(_j + 1) * _C] = 1.0
_COLF_NP = np.tile(np.arange(_C, dtype=np.float32), 8)[None, :]


def _focal_kernel(x_ref, t0_ref, t1_ref, w0_ref, w1_ref, ex_ref, colf_ref, out_ref):
    i = pl.program_id(0)
    x = x_ref[0]
    t0 = t0_ref[0]
    t1 = t1_ref[0]
    w0 = w0_ref[0]
    w1 = w1_ref[0]

    e = jnp.exp(-jnp.abs(x))
    lp = jnp.minimum(x, 0.0) - jnp.log1p(e)
    l1mp = lp - x
    p = jnp.exp(lp)
    omp = 1.0 - p
    pos = (-_ALPHA) * (omp * omp) * lp
    neg = (-(1.0 - _ALPHA)) * (p * p) * l1mp

    ex = ex_ref[...]
    colf = colf_ref[...]
    t0e = jnp.dot(t0, ex, preferred_element_type=jnp.float32)
    t1e = jnp.dot(t1, ex, preferred_element_type=jnp.float32)
    e0 = t0e == colf
    e1 = t1e == colf

    mp1 = jnp.where(e1, pos, 0.0)
    mp0 = jnp.where(e1, 0.0, jnp.where(e0, pos, 0.0))
    mn = jnp.where(e0 | e1, 0.0, neg)

    f = ex.T
    u1 = jnp.dot(mp1, f, preferred_element_type=jnp.float32)
    u0 = jnp.dot(mp0, f, preferred_element_type=jnp.float32)
    v = jnp.dot(mn, f, preferred_element_type=jnp.float32)

    a0 = t0 < float(_C)
    a1 = t1 < float(_C)
    qb = jnp.where(a0 & ~a1, w0, jnp.where(a1 & ~a0, w1, 0.5 * (w0 + w1)))
    contrib = w1 * u1 + w0 * u0 + qb * v

    @pl.when(i == 0)
    def _():
        out_ref[...] = jnp.zeros_like(out_ref)

    out_ref[...] += jnp.sum(contrib, axis=(0, 1), keepdims=True)


def kernel(pred, target0, target1, weight0, weight1):
    n, c = pred.shape
    predv = pred.reshape(_G, _R, _L)
    t0 = target0.astype(jnp.float32).reshape(_G, _R, 8)
    t1 = target1.astype(jnp.float32).reshape(_G, _R, 8)
    w0 = weight0.reshape(_G, _R, 8)
    w1 = weight1.reshape(_G, _R, 8)
    out = pl.pallas_call(
        _focal_kernel,
        grid=(_G,),
        in_specs=[
            pl.BlockSpec((1, _R, _L), lambda i: (i, 0, 0)),
            pl.BlockSpec((1, _R, 8), lambda i: (i, 0, 0)),
            pl.BlockSpec((1, _R, 8), lambda i: (i, 0, 0)),
            pl.BlockSpec((1, _R, 8), lambda i: (i, 0, 0)),
            pl.BlockSpec((1, _R, 8), lambda i: (i, 0, 0)),
            pl.BlockSpec((8, _L), lambda i: (0, 0)),
            pl.BlockSpec((1, _L), lambda i: (0, 0)),
        ],
        out_specs=pl.BlockSpec((1, 1), lambda i: (0, 0)),
        out_shape=jax.ShapeDtypeStruct((1, 1), jnp.float32),
        compiler_params=pltpu.CompilerParams(
            dimension_semantics=("arbitrary",),
        ),
    )(predv, t0, t1, w0, w1, jnp.asarray(_E_NP), jnp.asarray(_COLF_NP))
    return out[0, 0] * (_LOSS_WEIGHT / (n * c))


# speedup vs baseline: 3.2878x; 2.2855x over previous
import jax
import jax.numpy as jnp
from jax.experimental import pallas as pl
from jax.experimental.pallas import tpu as pltpu

_N = 100000
_C = 80
_ALPHA = 0.25
_LOSS_WEIGHT = 1.0
_BR = 4096


def _focal_kernel(x_ref, t0_ref, t1_ref, w0_ref, w1_ref, out_ref):
    i = pl.program_id(0)
    limit = _N - i * _BR

    x = x_ref[...]
    rows2d = jax.lax.broadcasted_iota(jnp.int32, (_BR, _C), 0)
    x = jnp.where(rows2d < limit, x, 0.0)

    e = jnp.exp(-jnp.abs(x))
    lp = jnp.minimum(x, 0.0) - jnp.log1p(e)
    l1mp = lp - x
    p = jnp.exp(lp)
    omp = 1.0 - p
    pos = (-_ALPHA) * (omp * omp) * lp
    neg = (-(1.0 - _ALPHA)) * (p * p) * l1mp

    t0r = t0_ref[...].reshape(1, _BR)
    t1r = t1_ref[...].reshape(1, _BR)
    w0r = w0_ref[...].reshape(1, _BR)
    w1r = w1_ref[...].reshape(1, _BR)
    lanes = jax.lax.broadcasted_iota(jnp.int32, (1, _BR), 1)
    vm = lanes < limit
    a0 = t0r < _C
    a1 = t1r < _C
    w1m = jnp.where(vm, w1r, 0.0)
    w0m = jnp.where(vm, w0r, 0.0)
    qb = jnp.where(a0 & ~a1, w0r, jnp.where(a1 & ~a0, w1r, 0.5 * (w0r + w1r)))
    qbm = jnp.where(vm, qb, 0.0)

    cls = jax.lax.broadcasted_iota(jnp.int32, (_C, _BR), 0)
    e0t = jnp.broadcast_to(t0r, (_C, _BR)) == cls
    e1t = jnp.broadcast_to(t1r, (_C, _BR)) == cls
    coef_p_t = jnp.where(e1t, jnp.broadcast_to(w1m, (_C, _BR)),
                         jnp.where(e0t, jnp.broadcast_to(w0m, (_C, _BR)), 0.0))
    coef_n_t = jnp.where(e0t | e1t, 0.0, jnp.broadcast_to(qbm, (_C, _BR)))

    cp = jnp.dot(coef_p_t, pos, preferred_element_type=jnp.float32)
    cn = jnp.dot(coef_n_t, neg, preferred_element_type=jnp.float32)
    cc = cp + cn
    dr = jax.lax.broadcasted_iota(jnp.int32, (_C, _C), 0)
    dc = jax.lax.broadcasted_iota(jnp.int32, (_C, _C), 1)
    diag = jnp.where(dr == dc, cc, 0.0)

    @pl.when(i == 0)
    def _():
        out_ref[...] = jnp.zeros_like(out_ref)

    out_ref[...] += jnp.sum(diag, axis=(0, 1), keepdims=True)


def kernel(pred, target0, target1, weight0, weight1):
    n, c = pred.shape
    grid = pl.cdiv(n, _BR)
    out = pl.pallas_call(
        _focal_kernel,
        grid=(grid,),
        in_specs=[
            pl.BlockSpec((_BR, c), lambda i: (i, 0)),
            pl.BlockSpec((_BR,), lambda i: (i,)),
            pl.BlockSpec((_BR,), lambda i: (i,)),
            pl.BlockSpec((_BR,), lambda i: (i,)),
            pl.BlockSpec((_BR,), lambda i: (i,)),
        ],
        out_specs=pl.BlockSpec((1, 1), lambda i: (0, 0)),
        out_shape=jax.ShapeDtypeStruct((1, 1), jnp.float32),
        compiler_params=pltpu.CompilerParams(
            dimension_semantics=("arbitrary",),
        ),
    )(pred, target0, target1, weight0, weight1)
    return out[0, 0] * (_LOSS_WEIGHT / (n * c))
